# baseline (speedup 1.0000x reference)
import functools

import jax
import jax.numpy as jnp
from jax import lax
from jax.experimental import pallas as pl
from jax.experimental.pallas import tpu as pltpu
from jax.experimental.pallas import tpu_sc as plsc

NC, NS, L = 2, 16, 16
NW = NC * NS

NA = 10000
NB = 160000
MAXNB = 16
H = 600
HP = 640
FA = 133
FB = 147
DEPTH = 6

NA_PAD = 10240
A_PER_W = NA_PAD // NW
SEG_CA = 4
B_PER_W = NB // NW
GD_CB = 40

_mesh = plsc.VectorSubcoreMesh(core_axis_name="c", subcore_axis_name="s")


@functools.partial(
    pl.kernel,
    out_type=jax.ShapeDtypeStruct((NA_PAD, HP), jnp.float32),
    mesh=_mesh,
    scratch_types=[
        pltpu.VMEM((SEG_CA * MAXNB,), jnp.int32),
        pltpu.VMEM((SEG_CA * MAXNB, HP), jnp.float32),
        pltpu.VMEM((SEG_CA, HP), jnp.float32),
        pltpu.SemaphoreType.DMA,
    ],
)
def _sc_segsum(msg_hbm, idx_hbm, out_hbm, idx_v, rows_v, acc_v, sem):
    wid = lax.axis_index("s") * NC + lax.axis_index("c")

    def body(ci, carry):
        abase = wid * A_PER_W + ci * SEG_CA
        pltpu.sync_copy(idx_hbm.at[pl.ds(abase * MAXNB, SEG_CA * MAXNB)], idx_v)
        pltpu.async_copy(msg_hbm.at[idx_v], rows_v, sem).wait()
        for a in range(SEG_CA):
            for c in range(HP // L):
                s = rows_v[a * MAXNB, pl.ds(c * L, L)]
                for j in range(1, MAXNB):
                    s = s + rows_v[a * MAXNB + j, pl.ds(c * L, L)]
                acc_v[a, pl.ds(c * L, L)] = s
        pltpu.sync_copy(acc_v, out_hbm.at[pl.ds(abase, SEG_CA)])
        return carry

    lax.fori_loop(0, A_PER_W // SEG_CA, body, 0)


@functools.partial(
    pl.kernel,
    out_type=jax.ShapeDtypeStruct((NB, HP), jnp.float32),
    mesh=_mesh,
    scratch_types=[
        pltpu.VMEM((GD_CB,), jnp.int32),
        pltpu.VMEM((GD_CB,), jnp.int32),
        pltpu.VMEM((GD_CB, HP), jnp.float32),
        pltpu.VMEM((GD_CB, HP), jnp.float32),
        pltpu.SemaphoreType.DMA,
    ],
)
def _sc_gather_diff(am_hbm, msg_hbm, b2a_hbm, b2revb_hbm, out_hbm,
                    ia_v, ir_v, arow_v, mrow_v, sem):
    wid = lax.axis_index("s") * NC + lax.axis_index("c")

    def body(ci, carry):
        bbase = wid * B_PER_W + ci * GD_CB
        pltpu.sync_copy(b2a_hbm.at[pl.ds(bbase, GD_CB)], ia_v)
        pltpu.sync_copy(b2revb_hbm.at[pl.ds(bbase, GD_CB)], ir_v)
        pltpu.async_copy(am_hbm.at[ia_v], arow_v, sem).wait()
        pltpu.async_copy(msg_hbm.at[ir_v], mrow_v, sem).wait()

        def row_body(r, carry2):
            for c in range(HP // L):
                arow_v[r, pl.ds(c * L, L)] = (
                    arow_v[r, pl.ds(c * L, L)] - mrow_v[r, pl.ds(c * L, L)])
            return carry2

        lax.fori_loop(0, GD_CB, row_body, 0)
        pltpu.sync_copy(arow_v, out_hbm.at[pl.ds(bbase, GD_CB)])
        return carry

    lax.fori_loop(0, B_PER_W // GD_CB, body, 0)


BM = 800
BMA = 400


def _mm_in_body(fb_ref, w_ref, inp_ref, msg_ref):
    x = jnp.dot(fb_ref[...], w_ref[...], preferred_element_type=jnp.float32)
    inp_ref[...] = x
    msg_ref[...] = jnp.maximum(x, 0.0)


def _tc_in(f_bonds, w_i_t):
    return pl.pallas_call(
        _mm_in_body,
        grid=(NB // BM,),
        in_specs=[
            pl.BlockSpec((BM, FB), lambda i: (i, 0)),
            pl.BlockSpec((FB, HP), lambda i: (0, 0)),
        ],
        out_specs=[
            pl.BlockSpec((BM, HP), lambda i: (i, 0)),
            pl.BlockSpec((BM, HP), lambda i: (i, 0)),
        ],
        out_shape=[
            jax.ShapeDtypeStruct((NB, HP), jnp.float32),
            jax.ShapeDtypeStruct((NB, HP), jnp.float32),
        ],
    )(f_bonds, w_i_t)


def _mm_h_body(t_ref, w_ref, inp_ref, out_ref):
    x = jnp.dot(t_ref[...], w_ref[...], preferred_element_type=jnp.float32)
    out_ref[...] = jnp.maximum(inp_ref[...] + x, 0.0)


def _tc_h(t, w_h_t, inp):
    return pl.pallas_call(
        _mm_h_body,
        grid=(NB // BM,),
        in_specs=[
            pl.BlockSpec((BM, HP), lambda i: (i, 0)),
            pl.BlockSpec((HP, HP), lambda i: (0, 0)),
            pl.BlockSpec((BM, HP), lambda i: (i, 0)),
        ],
        out_specs=pl.BlockSpec((BM, HP), lambda i: (i, 0)),
        out_shape=jax.ShapeDtypeStruct((NB, HP), jnp.float32),
    )(t, w_h_t, inp)


def _readout_body(fa_ref, am_ref, woa_ref, woh_ref, bo_ref,
                  w1_ref, b1_ref, w2_ref, b2_ref, wr_ref, br_ref, out_ref):
    x = jnp.dot(fa_ref[...], woa_ref[...], preferred_element_type=jnp.float32)
    x = x + jnp.dot(am_ref[...][:, :H], woh_ref[...],
                    preferred_element_type=jnp.float32)
    x = jnp.maximum(x + bo_ref[...], 0.0)
    h = jnp.maximum(
        jnp.dot(x, w1_ref[...], preferred_element_type=jnp.float32)
        + b1_ref[...], 0.0)
    h = jnp.maximum(
        jnp.dot(h, w2_ref[...], preferred_element_type=jnp.float32)
        + b2_ref[...], 0.0)
    o = jnp.dot(h, wr_ref[...], preferred_element_type=jnp.float32) + br_ref[...]
    out_ref[...] = jax.nn.sigmoid(o)


def _tc_readout(f_atoms, am, woa_t, woh_t, b_o, w1_t, b1, w2_t, b2, wr_t, br):
    return pl.pallas_call(
        _readout_body,
        grid=(NA // BMA,),
        in_specs=[
            pl.BlockSpec((BMA, FA), lambda i: (i, 0)),
            pl.BlockSpec((BMA, HP), lambda i: (i, 0)),
            pl.BlockSpec((FA, H), lambda i: (0, 0)),
            pl.BlockSpec((H, H), lambda i: (0, 0)),
            pl.BlockSpec((1, H), lambda i: (0, 0)),
            pl.BlockSpec((H, H), lambda i: (0, 0)),
            pl.BlockSpec((1, H), lambda i: (0, 0)),
            pl.BlockSpec((H, H), lambda i: (0, 0)),
            pl.BlockSpec((1, H), lambda i: (0, 0)),
            pl.BlockSpec((H, 1), lambda i: (0, 0)),
            pl.BlockSpec((1, 1), lambda i: (0, 0)),
        ],
        out_specs=pl.BlockSpec((BMA, 1), lambda i: (i, 0)),
        out_shape=jax.ShapeDtypeStruct((NA, 1), jnp.float32),
    )(f_atoms, am, woa_t, woh_t, b_o, w1_t, b1, w2_t, b2, wr_t, br)


def kernel(f_atoms, f_bonds, a2b, b2a, b2revb,
           W_i, W_h, W_o, b_o, W_f1, b_f1, W_f2, b_f2, W_r, b_r):
    w_i_t = jnp.zeros((FB, HP), jnp.float32).at[:, :H].set(W_i.T)
    w_h_t = jnp.zeros((HP, HP), jnp.float32).at[:H, :H].set(W_h.T)
    a2b_flat = jnp.pad(a2b, ((0, NA_PAD - NA), (0, 0))).reshape(-1)

    inp, msg = _tc_in(f_bonds, w_i_t)
    for _ in range(DEPTH - 1):
        am = _sc_segsum(msg, a2b_flat)
        t = _sc_gather_diff(am, msg, b2a, b2revb)
        msg = _tc_h(t, w_h_t, inp)
    am = _sc_segsum(msg, a2b_flat)

    out = _tc_readout(
        f_atoms, am,
        W_o[:, :FA].T, W_o[:, FA:].T, b_o.reshape(1, H),
        W_f1.T, b_f1.reshape(1, H), W_f2.T, b_f2.reshape(1, H),
        W_r.T, b_r.reshape(1, 1))
    return out[1:]


# speedup vs baseline: 1.4734x; 1.4734x over previous
import functools

import jax
import jax.numpy as jnp
from jax import lax
from jax.experimental import pallas as pl
from jax.experimental.pallas import tpu as pltpu
from jax.experimental.pallas import tpu_sc as plsc

NC, NS, L = 2, 16, 16
NW = NC * NS

NA = 10000
NB = 160000
MAXNB = 16
H = 600
HP = 640
FA = 133
FB = 147
DEPTH = 6

NA_PAD = 10240
A_PER_W = NA_PAD // NW
SEG_CA = 4
B_PER_W = NB // NW
GD_CB = 40

_mesh = plsc.VectorSubcoreMesh(core_axis_name="c", subcore_axis_name="s")


SEG_NCH = A_PER_W // SEG_CA
GD_NCH = B_PER_W // GD_CB


@functools.partial(
    pl.kernel,
    out_type=jax.ShapeDtypeStruct((NA_PAD, HP), jnp.float32),
    mesh=_mesh,
    scratch_types=[
        pltpu.VMEM((SEG_CA * MAXNB,), jnp.int32),
        pltpu.VMEM((SEG_CA * MAXNB,), jnp.int32),
        pltpu.VMEM((SEG_CA * MAXNB, HP), jnp.float32),
        pltpu.VMEM((SEG_CA * MAXNB, HP), jnp.float32),
        pltpu.VMEM((SEG_CA, HP), jnp.float32),
        pltpu.SemaphoreType.DMA,
        pltpu.SemaphoreType.DMA,
    ],
)
def _sc_segsum(msg_hbm, idx_hbm, out_hbm, idx0, idx1, rows0, rows1, acc_v,
               sem0, sem1):
    wid = lax.axis_index("s") * NC + lax.axis_index("c")
    idxs, rows, sems = (idx0, idx1), (rows0, rows1), (sem0, sem1)

    def fire(ci, p):
        abase = wid * A_PER_W + ci * SEG_CA
        pltpu.sync_copy(idx_hbm.at[pl.ds(abase * MAXNB, SEG_CA * MAXNB)],
                        idxs[p])
        pltpu.async_copy(msg_hbm.at[idxs[p]], rows[p], sems[p])

    def consume(ci, p):
        pltpu.make_async_copy(msg_hbm.at[idxs[p]], rows[p], sems[p]).wait()
        rp = rows[p]

        def atom_body(a, carry):
            base = a * MAXNB
            for c in range(HP // L):
                s = rp[base, pl.ds(c * L, L)]
                for j in range(1, MAXNB):
                    s = s + rp[base + j, pl.ds(c * L, L)]
                acc_v[a, pl.ds(c * L, L)] = s
            return carry

        lax.fori_loop(0, SEG_CA, atom_body, 0)
        pltpu.sync_copy(acc_v, out_hbm.at[pl.ds(wid * A_PER_W + ci * SEG_CA,
                                                SEG_CA)])

    fire(0, 0)
    fire(1, 1)

    def body(cj, carry):
        ci = 2 * cj
        consume(ci, 0)
        fire(ci + 2, 0)
        consume(ci + 1, 1)
        fire(ci + 3, 1)
        return carry

    lax.fori_loop(0, SEG_NCH // 2 - 1, body, 0)
    consume(SEG_NCH - 2, 0)
    consume(SEG_NCH - 1, 1)


@functools.partial(
    pl.kernel,
    out_type=jax.ShapeDtypeStruct((NB, HP), jnp.float32),
    mesh=_mesh,
    scratch_types=[
        pltpu.VMEM((2 * GD_CB,), jnp.int32),
        pltpu.VMEM((2 * GD_CB,), jnp.int32),
        pltpu.VMEM((GD_CB, HP), jnp.float32),
        pltpu.VMEM((GD_CB, HP), jnp.float32),
        pltpu.VMEM((GD_CB, HP), jnp.float32),
        pltpu.VMEM((GD_CB, HP), jnp.float32),
        pltpu.SemaphoreType.DMA,
        pltpu.SemaphoreType.DMA,
        pltpu.SemaphoreType.DMA,
        pltpu.SemaphoreType.DMA,
    ],
)
def _sc_gather_diff(am_hbm, msg_hbm, idx2_hbm, out_hbm,
                    ii0, ii1, arow0, arow1, mrow0, mrow1,
                    semA0, semA1, semM0, semM1):
    wid = lax.axis_index("s") * NC + lax.axis_index("c")
    iis = (ii0, ii1)
    arows, mrows = (arow0, arow1), (mrow0, mrow1)
    semsA, semsM = (semA0, semA1), (semM0, semM1)

    def fire(ci, p):
        cg = wid * GD_NCH + ci
        pltpu.sync_copy(idx2_hbm.at[pl.ds(cg * 2 * GD_CB, 2 * GD_CB)], iis[p])
        pltpu.async_copy(am_hbm.at[iis[p].at[pl.ds(0, GD_CB)]],
                         arows[p], semsA[p])
        pltpu.async_copy(msg_hbm.at[iis[p].at[pl.ds(GD_CB, GD_CB)]],
                         mrows[p], semsM[p])

    def consume(ci, p):
        pltpu.make_async_copy(am_hbm.at[iis[p].at[pl.ds(0, GD_CB)]],
                              arows[p], semsA[p]).wait()
        pltpu.make_async_copy(msg_hbm.at[iis[p].at[pl.ds(GD_CB, GD_CB)]],
                              mrows[p], semsM[p]).wait()
        ap, mp = arows[p], mrows[p]

        def row_body(r, carry2):
            for c in range(HP // L):
                ap[r, pl.ds(c * L, L)] = (
                    ap[r, pl.ds(c * L, L)] - mp[r, pl.ds(c * L, L)])
            return carry2

        lax.fori_loop(0, GD_CB, row_body, 0)
        pltpu.sync_copy(
            ap, out_hbm.at[pl.ds(wid * B_PER_W + ci * GD_CB, GD_CB)])

    fire(0, 0)
    fire(1, 1)

    def body(cj, carry):
        ci = 2 * cj
        consume(ci, 0)
        fire(ci + 2, 0)
        consume(ci + 1, 1)
        fire(ci + 3, 1)
        return carry

    lax.fori_loop(0, (GD_NCH - 1) // 2 - 1, body, 0)
    consume(GD_NCH - 3, 0)
    fire(GD_NCH - 1, 0)
    consume(GD_NCH - 2, 1)
    consume(GD_NCH - 1, 0)


BM = 800
BMA = 400


def _mm_in_body(fb_ref, w_ref, inp_ref, msg_ref):
    x = jnp.dot(fb_ref[...], w_ref[...], preferred_element_type=jnp.float32)
    inp_ref[...] = x
    msg_ref[...] = jnp.maximum(x, 0.0)


def _tc_in(f_bonds, w_i_t):
    return pl.pallas_call(
        _mm_in_body,
        grid=(NB // BM,),
        in_specs=[
            pl.BlockSpec((BM, FB), lambda i: (i, 0)),
            pl.BlockSpec((FB, HP), lambda i: (0, 0)),
        ],
        out_specs=[
            pl.BlockSpec((BM, HP), lambda i: (i, 0)),
            pl.BlockSpec((BM, HP), lambda i: (i, 0)),
        ],
        out_shape=[
            jax.ShapeDtypeStruct((NB, HP), jnp.float32),
            jax.ShapeDtypeStruct((NB, HP), jnp.float32),
        ],
    )(f_bonds, w_i_t)


def _mm_h_body(t_ref, w_ref, inp_ref, out_ref):
    x = jnp.dot(t_ref[...], w_ref[...], preferred_element_type=jnp.float32)
    out_ref[...] = jnp.maximum(inp_ref[...] + x, 0.0)


def _tc_h(t, w_h_t, inp):
    return pl.pallas_call(
        _mm_h_body,
        grid=(NB // BM,),
        in_specs=[
            pl.BlockSpec((BM, HP), lambda i: (i, 0)),
            pl.BlockSpec((HP, HP), lambda i: (0, 0)),
            pl.BlockSpec((BM, HP), lambda i: (i, 0)),
        ],
        out_specs=pl.BlockSpec((BM, HP), lambda i: (i, 0)),
        out_shape=jax.ShapeDtypeStruct((NB, HP), jnp.float32),
    )(t, w_h_t, inp)


def _readout_body(fa_ref, am_ref, woa_ref, woh_ref, bo_ref,
                  w1_ref, b1_ref, w2_ref, b2_ref, wr_ref, br_ref, out_ref):
    x = jnp.dot(fa_ref[...], woa_ref[...], preferred_element_type=jnp.float32)
    x = x + jnp.dot(am_ref[...][:, :H], woh_ref[...],
                    preferred_element_type=jnp.float32)
    x = jnp.maximum(x + bo_ref[...], 0.0)
    h = jnp.maximum(
        jnp.dot(x, w1_ref[...], preferred_element_type=jnp.float32)
        + b1_ref[...], 0.0)
    h = jnp.maximum(
        jnp.dot(h, w2_ref[...], preferred_element_type=jnp.float32)
        + b2_ref[...], 0.0)
    o = jnp.dot(h, wr_ref[...], preferred_element_type=jnp.float32) + br_ref[...]
    out_ref[...] = jax.nn.sigmoid(o)


def _tc_readout(f_atoms, am, woa_t, woh_t, b_o, w1_t, b1, w2_t, b2, wr_t, br):
    return pl.pallas_call(
        _readout_body,
        grid=(NA // BMA,),
        in_specs=[
            pl.BlockSpec((BMA, FA), lambda i: (i, 0)),
            pl.BlockSpec((BMA, HP), lambda i: (i, 0)),
            pl.BlockSpec((FA, H), lambda i: (0, 0)),
            pl.BlockSpec((H, H), lambda i: (0, 0)),
            pl.BlockSpec((1, H), lambda i: (0, 0)),
            pl.BlockSpec((H, H), lambda i: (0, 0)),
            pl.BlockSpec((1, H), lambda i: (0, 0)),
            pl.BlockSpec((H, H), lambda i: (0, 0)),
            pl.BlockSpec((1, H), lambda i: (0, 0)),
            pl.BlockSpec((H, 1), lambda i: (0, 0)),
            pl.BlockSpec((1, 1), lambda i: (0, 0)),
        ],
        out_specs=pl.BlockSpec((BMA, 1), lambda i: (i, 0)),
        out_shape=jax.ShapeDtypeStruct((NA, 1), jnp.float32),
    )(f_atoms, am, woa_t, woh_t, b_o, w1_t, b1, w2_t, b2, wr_t, br)


def kernel(f_atoms, f_bonds, a2b, b2a, b2revb,
           W_i, W_h, W_o, b_o, W_f1, b_f1, W_f2, b_f2, W_r, b_r):
    w_i_t = jnp.zeros((FB, HP), jnp.float32).at[:, :H].set(W_i.T)
    w_h_t = jnp.zeros((HP, HP), jnp.float32).at[:H, :H].set(W_h.T)
    a2b_flat = jnp.pad(a2b, ((0, NA_PAD - NA), (0, 0))).reshape(-1)
    idx2 = jnp.stack([b2a.reshape(NB // GD_CB, GD_CB),
                      b2revb.reshape(NB // GD_CB, GD_CB)], axis=1).reshape(-1)

    inp, msg = _tc_in(f_bonds, w_i_t)
    for _ in range(DEPTH - 1):
        am = _sc_segsum(msg, a2b_flat)
        t = _sc_gather_diff(am, msg, idx2)
        msg = _tc_h(t, w_h_t, inp)
    am = _sc_segsum(msg, a2b_flat)

    out = _tc_readout(
        f_atoms, am,
        W_o[:, :FA].T, W_o[:, FA:].T, b_o.reshape(1, H),
        W_f1.T, b_f1.reshape(1, H), W_f2.T, b_f2.reshape(1, H),
        W_r.T, b_r.reshape(1, 1))
    return out[1:]
